# baseline (speedup 1.0000x reference)
import functools

import jax
import jax.numpy as jnp
from jax import lax
from jax.experimental import pallas as pl
from jax.experimental.pallas import tpu as pltpu
from jax.experimental.pallas import tpu_sc as plsc

N = 10000
E = 320000
F = 128
NSUB = 4096
S = 16
M = NSUB * S
K2 = 2 * M
NCQ = 3
H = 1024
NCORE = 2
NSC = 16
NW = NCORE * NSC

_f32 = jnp.float32
_i32 = jnp.int32

_mesh = plsc.VectorSubcoreMesh(core_axis_name="c", subcore_axis_name="s")
_sc_params = pltpu.CompilerParams(needs_layout_passes=False)
_sc_params_nt = pltpu.CompilerParams(needs_layout_passes=False,
                                     use_tc_tiling_on_sc=False)


def _leaky(x):
    return jnp.maximum(x, x * 0.01)


def _z16f():
    return jnp.zeros((16,), _f32)


_EB1 = 2560


def _sc_scatter_body(xt, src_h, dst_h, a_out, xblk, ablk, srcb, dstb):
    c = lax.axis_index("c")
    s = lax.axis_index("s")
    w = c * NSC + s
    pltpu.sync_copy(xt.at[w], xblk)

    def zero(i, _):
        ablk[pl.ds(i * 16, 16)] = _z16f()
        return 0

    lax.fori_loop(0, (N * 4) // 16, zero, 0)

    def blk(b, _):
        pltpu.sync_copy(src_h.at[pl.ds(b * _EB1, _EB1)], srcb)
        pltpu.sync_copy(dst_h.at[pl.ds(b * _EB1, _EB1)], dstb)

        def inner(i, _):
            for u in range(2):
                s16 = srcb[pl.ds(i * 32 + u * 16, 16)] * 4
                d16 = dstb[pl.ds(i * 32 + u * 16, 16)] * 4
                for j in range(4):
                    v = plsc.load_gather(xblk, [s16 + j])
                    plsc.addupdate_scatter(ablk, [d16 + j], v)
            return 0

        lax.fori_loop(0, _EB1 // 32, inner, 0)
        return 0

    lax.fori_loop(0, E // _EB1, blk, 0)
    pltpu.sync_copy(ablk, a_out.at[w])


_sc_scatter = pl.kernel(
    _sc_scatter_body,
    out_type=jax.ShapeDtypeStruct((NW, N * 4), _f32),
    mesh=_mesh,
    compiler_params=_sc_params,
    scratch_types=[
        pltpu.VMEM((N * 4,), _f32),
        pltpu.VMEM((N * 4,), _f32),
        pltpu.VMEM((_EB1,), _i32),
        pltpu.VMEM((_EB1,), _i32),
    ],
)


def _tc_c_body(a_ref, x_ref, ws1, wd1, b1, ws2, wd2, b2, c1_ref, c2_ref):
    a = a_ref[...]
    x = x_ref[...]
    c1_ref[...] = (jnp.dot(a, ws1[...], preferred_element_type=_f32,
                           precision=lax.Precision.HIGHEST)
                   + jnp.dot(x, wd1[...], preferred_element_type=_f32,
                           precision=lax.Precision.HIGHEST)
                   + b1[...])
    c2_ref[...] = (jnp.dot(a, ws2[...], preferred_element_type=_f32,
                           precision=lax.Precision.HIGHEST)
                   + jnp.dot(x, wd2[...], preferred_element_type=_f32,
                           precision=lax.Precision.HIGHEST)
                   + b2[...])


def _tc_c(a, x, ws1, wd1, b1, ws2, wd2, b2):
    bn = 1000
    row = pl.BlockSpec((bn, F), lambda i: (i, 0))
    wsp = pl.BlockSpec((F, F), lambda i: (0, 0))
    bsp = pl.BlockSpec((1, F), lambda i: (0, 0))
    return pl.pallas_call(
        _tc_c_body,
        grid=(N // bn,),
        in_specs=[row, row, wsp, wsp, bsp, wsp, wsp, bsp],
        out_specs=[row, row],
        out_shape=[jax.ShapeDtypeStruct((N, F), _f32)] * 2,
    )(a, x, ws1, wd1, b1, ws2, wd2, b2)


_EB2 = 2560


def _sc_stats_body(ctall, wall, dst_h, ang_h, act_h, gte_h,
                   st_out, cblk, dstb, ab, qb, gb, wav, wqv, wgv, accv):
    c = lax.axis_index("c")
    s = lax.axis_index("s")
    pltpu.sync_copy(ctall.at[c * NSC + s], cblk)
    pltpu.sync_copy(wall.at[c * 3 + 0], wav)
    pltpu.sync_copy(wall.at[c * 3 + 1], wqv)
    pltpu.sync_copy(wall.at[c * 3 + 2], wgv)

    base = s * 8
    waf = [plsc.load_gather(wav, [jnp.full((16,), base + j, _i32)])
           for j in range(8)]
    wqf = [plsc.load_gather(wqv, [jnp.full((16,), base + j, _i32)])
           for j in range(8)]
    wgf = [plsc.load_gather(wgv, [jnp.full((16,), base + j, _i32)])
           for j in range(8)]

    def blk(b, carry):
        pltpu.sync_copy(dst_h.at[pl.ds(b * _EB2, _EB2)], dstb)
        pltpu.sync_copy(ang_h.at[pl.ds(b * _EB2, _EB2)], ab)
        pltpu.sync_copy(act_h.at[pl.ds(b * _EB2, _EB2)], qb)
        pltpu.sync_copy(gte_h.at[pl.ds(b * _EB2, _EB2)], gb)

        def inner(i, cr):
            sums = list(cr[:8])
            sqsl = list(cr[8:])
            for u in range(2):
                d16 = dstb[pl.ds(i * 32 + u * 16, 16)] * 8
                a16 = ab[pl.ds(i * 32 + u * 16, 16)]
                q16 = qb[pl.ds(i * 32 + u * 16, 16)]
                g16 = gb[pl.ds(i * 32 + u * 16, 16)]
                for j in range(8):
                    cj = plsc.load_gather(cblk, [d16 + j])
                    h = cj + a16 * waf[j] + q16 * wqf[j] + g16 * wgf[j]
                    v = _leaky(h)
                    sums[j] = sums[j] + v
                    sqsl[j] = sqsl[j] + v * v
            return tuple(sums) + tuple(sqsl)

        part = lax.fori_loop(0, _EB2 // 32, inner,
                             tuple(_z16f() for _ in range(16)))
        return tuple(c + p for c, p in zip(carry, part))

    init = tuple(_z16f() for _ in range(16))
    acc = lax.fori_loop(0, E // _EB2, blk, init)
    for j in range(16):
        sj = jnp.full((16,), jnp.sum(acc[j]))
        plsc.store_scatter(accv, [jnp.full((16,), j, _i32)], sj)
    pltpu.sync_copy(accv, st_out.at[c, pl.ds(s * 16, 16)])


_sc_stats = pl.kernel(
    _sc_stats_body,
    out_type=jax.ShapeDtypeStruct((2, 256), _f32),
    mesh=_mesh,
    compiler_params=_sc_params,
    scratch_types=[
        pltpu.VMEM((N * 8,), _f32),
        pltpu.VMEM((_EB2,), _i32),
        pltpu.VMEM((_EB2,), _f32),
        pltpu.VMEM((_EB2,), _f32),
        pltpu.VMEM((_EB2,), _f32),
        pltpu.VMEM((F,), _f32),
        pltpu.VMEM((F,), _f32),
        pltpu.VMEM((F,), _f32),
        pltpu.VMEM((16,), _f32),
    ],
)


_MT = M // NW
_GB = 128


def _sc_fbuild_body(c1_h, c2_h, pe_h, sub_h, wpk_h,
                    fall_out,
                    sgb, peb, dlist, c1b, c2b, ob1, ob2, wpk, sem):
    c = lax.axis_index("c")
    s = lax.axis_index("s")
    w = c * NSC + s
    base = w * _MT
    pltpu.sync_copy(sub_h.at[pl.ds(base, _MT)], sgb)
    pltpu.sync_copy(wpk_h, wpk)
    descs = [
        pltpu.async_copy(pe_h.at[sgb.at[pl.ds(ch * _GB, _GB)]],
                         peb.at[pl.ds(ch * _GB, _GB)], sem)
        for ch in range(_MT // _GB)
    ]
    for d in descs:
        d.wait()

    lane = jax.lax.iota(_i32, 16)

    def blk(b, _):
        for t in range(_GB // 16):
            r16 = b * _GB + t * 16 + lane
            dstf = plsc.load_gather(peb, [r16, jnp.zeros((16,), _i32)])
            dlist[pl.ds(t * 16, 16)] = plsc.bitcast(dstf, _i32)
        d1 = pltpu.async_copy(c1_h.at[dlist], c1b, sem)
        d2 = pltpu.async_copy(c2_h.at[dlist], c2b, sem)
        d1.wait()
        d2.wait()

        for lp in range(2):
            cb = (c1b, c2b)[lp]
            ob = (ob1, ob2)[lp]
            wv = [[wpk[3 * lp + r, pl.ds(j * 16, 16)] for j in range(8)]
                  for r in range(3)]

            def edge(i, _):
                row = b * _GB + i
                a16 = plsc.load_gather(
                    peb, [jnp.full((16,), row, _i32), jnp.full((16,), 1, _i32)])
                q16 = plsc.load_gather(
                    peb, [jnp.full((16,), row, _i32), jnp.full((16,), 2, _i32)])
                g16 = plsc.load_gather(
                    peb, [jnp.full((16,), row, _i32), jnp.full((16,), 3, _i32)])
                for j in range(8):
                    cj = cb[i, pl.ds(j * 16, 16)]
                    h = cj + a16 * wv[0][j] + q16 * wv[1][j] + g16 * wv[2][j]
                    ob[i, pl.ds(j * 16, 16)] = _leaky(h)
                return 0

            lax.fori_loop(0, _GB, edge, 0)

        rb = base + b * _GB
        pltpu.sync_copy(ob1, fall_out.at[pl.ds(rb, _GB)])
        pltpu.sync_copy(ob2, fall_out.at[pl.ds(M + rb, _GB)])
        return 0

    lax.fori_loop(0, _MT // _GB, blk, 0)


def _edge_rows_spec(shape):
    return pltpu.VMEM(shape, _f32)


_sc_fbuild = pl.kernel(
    _sc_fbuild_body,
    out_type=jax.ShapeDtypeStruct((2 * M, F), _f32),
    mesh=_mesh,
    compiler_params=_sc_params_nt,
    scratch_types=[
        pltpu.VMEM((_MT,), _i32),
        pltpu.VMEM((_MT, 16), _f32),
        pltpu.VMEM((_GB,), _i32),
        pltpu.VMEM((_GB, F), _f32),
        pltpu.VMEM((_GB, F), _f32),
        pltpu.VMEM((_GB, F), _f32),
        pltpu.VMEM((_GB, F), _f32),
        pltpu.VMEM((8, F), _f32),
        pltpu.SemaphoreType.DMA,
    ],
)


_DT = K2 // NW


def _sc_deg_body(dst_h, deg_out, degb, sepb):
    c = lax.axis_index("c")
    s = lax.axis_index("s")
    w = c * NSC + s

    def zero(i, _):
        degb[pl.ds(i * 16, 16)] = _z16f()
        return 0

    lax.fori_loop(0, M // 16, zero, 0)
    pltpu.sync_copy(dst_h.at[pl.ds(w * _DT, _DT)], sepb)
    ones = jnp.ones((16,), _f32)

    def hist(i, _):
        d16 = sepb[pl.ds(i * 16, 16)]
        plsc.addupdate_scatter(degb, [d16], ones)
        return 0

    lax.fori_loop(0, _DT // 16, hist, 0)
    pltpu.sync_copy(degb, deg_out.at[w])


_sc_deg = pl.kernel(
    _sc_deg_body,
    out_type=jax.ShapeDtypeStruct((NW, M), _f32),
    mesh=_mesh,
    compiler_params=_sc_params,
    scratch_types=[
        pltpu.VMEM((M,), _f32),
        pltpu.VMEM((_DT,), _i32),
    ],
)


_AT = K2 // NSC
_CH = 4096
_CHP = _CH + 256
_NP = M // _CH


def _sc_agg_body(fall_h, src_h, dst_h, aggall_out,
                 srcb, dstb, selS, selD, gsrc, gdst, gsrc2, gdst2,
                 rowbuf, rowbuf2, zb, sem, acc):
    c = lax.axis_index("c")
    s = lax.axis_index("s")
    ebase = s * _AT
    pltpu.sync_copy(src_h.at[pl.ds(ebase, _AT)], srcb)
    pltpu.sync_copy(dst_h.at[pl.ds(ebase, _AT)], dstb)

    def zzb(k, _):
        for t in range(8):
            zb[k, pl.ds(t * 16, 16)] = _z16f()
        return 0

    lax.fori_loop(0, 16, zzb, 0)

    rows_per_tile = _CHP // NSC

    for p in range(_NP):
        def zr(k, _):
            pltpu.sync_copy(zb, acc.at[pl.ds(s * rows_per_tile + k * 16, 16)])
            return 0

        lax.fori_loop(0, rows_per_tile // 16, zr, 0)

        plsc.subcore_barrier()

        lo = p * _CH

        def scan(i, off):
            d16 = dstb[pl.ds(i * 16, 16)] - lo
            s16 = srcb[pl.ds(i * 16, 16)]
            m = (d16 >= 0) & (d16 < _CH)
            plsc.store_compressed(selS.at[pl.ds(off, 16)], s16, mask=m)
            plsc.store_compressed(selD.at[pl.ds(off, 16)], d16, mask=m)
            return off + jnp.sum(jnp.where(m, 1, 0))

        off = lax.fori_loop(0, _AT // 16, scan, 0)

        def pad(k, _):
            selD[pl.ds(off + k * 16, 16)] = jnp.full((16,), _CH, _i32)
            selS[pl.ds(off + k * 16, 16)] = jnp.zeros((16,), _i32)
            return 0

        lax.fori_loop(0, 2 * _GB // 16, pad, 0)
        nblk = (off + _GB - 1) // _GB

        coff = c * M

        def gs(g, _):
            for t in range(_GB // 16):
                sl = pl.ds(t * 16, 16)
                gsrc[sl] = selS[pl.ds(g * 2 * _GB + t * 16, 16)] + coff
                gdst[sl] = selD[pl.ds(g * 2 * _GB + t * 16, 16)]
                gsrc2[sl] = selS[pl.ds((g * 2 + 1) * _GB + t * 16, 16)] + coff
                gdst2[sl] = selD[pl.ds((g * 2 + 1) * _GB + t * 16, 16)]
            d0 = pltpu.async_copy(fall_h.at[gsrc], rowbuf, sem)
            d1 = pltpu.async_copy(fall_h.at[gsrc2], rowbuf2, sem)
            d0.wait()
            pltpu.sync_copy(rowbuf, acc.at[gdst], add=True)
            d1.wait()
            pltpu.sync_copy(rowbuf2, acc.at[gdst2], add=True)
            return 0

        lax.fori_loop(0, (nblk + 1) // 2, gs, 0)
        plsc.subcore_barrier()

        wrow = s * (_CH // NSC)

        def wb(k, _):
            rsl = pl.ds(wrow + k * _GB, _GB)
            pltpu.sync_copy(acc.at[rsl], rowbuf)
            pltpu.sync_copy(
                rowbuf,
                aggall_out.at[pl.ds(c * M + lo + wrow + k * _GB, _GB)])
            return 0

        lax.fori_loop(0, (_CH // NSC) // _GB, wb, 0)
        plsc.subcore_barrier()


_sc_agg = pl.kernel(
    _sc_agg_body,
    out_type=jax.ShapeDtypeStruct((2 * M, F), _f32),
    mesh=_mesh,
    compiler_params=_sc_params,
    scratch_types=[
        pltpu.VMEM((_AT,), _i32),
        pltpu.VMEM((_AT,), _i32),
        pltpu.VMEM((_AT + 2 * _GB,), _i32),
        pltpu.VMEM((_AT + 2 * _GB,), _i32),
        pltpu.VMEM((_GB,), _i32),
        pltpu.VMEM((_GB,), _i32),
        pltpu.VMEM((_GB,), _i32),
        pltpu.VMEM((_GB,), _i32),
        pltpu.VMEM((_GB, F), _f32),
        pltpu.VMEM((_GB, F), _f32),
        pltpu.VMEM((16, F), _f32),
        pltpu.SemaphoreType.DMA,
        pltpu.VMEM_SHARED((_CHP, F), _f32),
    ],
)


def _tc_fin_body(f1, f2, a1, a2, dgp, sums, sqs, gma, bta,
                 wg1, bg1, wg2, bg2,
                 v1a, v1ab, v1b, v1bb, v1c, v1cb,
                 v2a, v2ab, v2b, v2bb, v2c, v2cb,
                 q1_ref, q2_ref):
    mu = sums[...] / E
    var = sqs[...] / E - mu * mu
    r = gma[...] / jnp.sqrt(var + 1e-5)
    sh = bta[...] - mu * r
    deg = jnp.sum(dgp[...], axis=0)
    rec = 1.0 / (deg + 1.0)
    s2c = (1.0 + deg * rec)[:, None]

    def head(fv, av, ri, shi, wg, bg, va, vab, vb, vbb, vc, vcb, out_ref):
        t = ri * (fv[...] + av[...] * rec[:, None]) + shi * s2c
        g = _leaky(jnp.dot(t, wg[...], preferred_element_type=_f32,
                           precision=lax.Precision.HIGHEST) + bg[...])
        mrows = g.shape[0] // S
        gm = jnp.mean(g.reshape(mrows, S, F), axis=1)
        z = _leaky(jnp.dot(gm, va[...], preferred_element_type=_f32,
                           precision=lax.Precision.HIGHEST) + vab[...])
        z = _leaky(jnp.dot(z, vb[...], preferred_element_type=_f32,
                           precision=lax.Precision.HIGHEST) + vbb[...])
        out_ref[...] = jnp.dot(z, vc[...], preferred_element_type=_f32,
                           precision=lax.Precision.HIGHEST) + vcb[...]

    head(f1, a1, r[0:1, :], sh[0:1, :], wg1, bg1,
         v1a, v1ab, v1b, v1bb, v1c, v1cb, q1_ref)
    head(f2, a2, r[1:2, :], sh[1:2, :], wg2, bg2,
         v2a, v2ab, v2b, v2bb, v2c, v2cb, q2_ref)


def _tc_final(fall, aggall, degp, sums, sqs, gma, bta,
              wg1, bg1, wg2, bg2, v1, v2):
    R = 2048
    grid = (M // R,)
    row = pl.BlockSpec((R, F), lambda i: (i, 0))
    row2 = pl.BlockSpec((R, F), lambda i: (i + M // R, 0))
    dsp = pl.BlockSpec((NW, R), lambda i: (0, i))
    c2f = pl.BlockSpec((2, F), lambda i: (0, 0))
    c1f = pl.BlockSpec((1, F), lambda i: (0, 0))
    wsp = pl.BlockSpec((F, F), lambda i: (0, 0))
    vaspec = pl.BlockSpec((F, H), lambda i: (0, 0))
    vbspec = pl.BlockSpec((H, H), lambda i: (0, 0))
    vcspec = pl.BlockSpec((H, NCQ), lambda i: (0, 0))
    h1 = pl.BlockSpec((1, H), lambda i: (0, 0))
    c1n = pl.BlockSpec((1, NCQ), lambda i: (0, 0))
    osp = pl.BlockSpec((R // S, NCQ), lambda i: (i, 0))
    v1a, v1ab, v1b, v1bb, v1c, v1cb = v1
    v2a, v2ab, v2b, v2bb, v2c, v2cb = v2
    return pl.pallas_call(
        _tc_fin_body,
        grid=grid,
        in_specs=[row, row2, row, row2, dsp, c2f, c2f, c1f, c1f,
                  wsp, c1f, wsp, c1f,
                  vaspec, h1, vbspec, h1, vcspec, c1n,
                  vaspec, h1, vbspec, h1, vcspec, c1n],
        out_specs=[osp, osp],
        out_shape=[jax.ShapeDtypeStruct((NSUB, NCQ), _f32)] * 2,
    )(fall, fall, aggall, aggall, degp, sums, sqs, gma, bta,
      wg1, bg1, wg2, bg2,
      v1a, v1ab, v1b, v1bb, v1c, v1cb,
      v2a, v2ab, v2b, v2bb, v2c, v2cb)


def kernel(node_features, actions, edge_index, angles, sub_graphs,
           sep_subgraphs, gt_edges, post_input,
           Wsrc1, Wdst1, wa1, wq1, wg1, b1,
           Wsrc2, Wdst2, wa2, wq2, wg2, b2,
           gamma, beta, Wg1, bg1, Wg2, bg2,
           V1a, V1ab, V1b, V1bb, V1c, V1cb,
           V2a, V2ab, V2b, V2bb, V2c, V2cb):
    x = node_features.astype(_f32)
    src = edge_index[0].astype(_i32)
    dst = edge_index[1].astype(_i32)
    sub = sub_graphs.astype(_i32)

    xt = x.reshape(N, NW, 4).transpose(1, 0, 2).reshape(NW, N * 4)
    a_out = _sc_scatter(xt, src, dst)
    A = a_out.reshape(NW, N, 4).transpose(1, 0, 2).reshape(N, F)

    c1, c2 = _tc_c(A, x, Wsrc1, Wdst1, b1.reshape(1, F),
                   Wsrc2, Wdst2, b2.reshape(1, F))

    ct1 = c1.reshape(N, NSC, 8).transpose(1, 0, 2).reshape(NSC, N * 8)
    ct2 = c2.reshape(N, NSC, 8).transpose(1, 0, 2).reshape(NSC, N * 8)
    ctall = jnp.concatenate([ct1, ct2], axis=0)
    wall = jnp.stack([wa1, wq1, wg1, wa2, wq2, wg2])
    st = _sc_stats(ctall, wall, dst, angles, actions, gt_edges)
    stv = st.reshape(2, NSC, 2, 8)
    sums = stv[:, :, 0, :].reshape(2, F)
    sqs = stv[:, :, 1, :].reshape(2, F)

    dstf = lax.bitcast_convert_type(dst, _f32)
    pe = jnp.concatenate(
        [dstf[:, None], angles[:, None], actions[:, None], gt_edges[:, None],
         jnp.zeros((E, 12), _f32)], axis=1)
    wpk = jnp.concatenate(
        [wa1[None], wq1[None], wg1[None], wa2[None], wq2[None], wg2[None],
         jnp.zeros((2, F), _f32)], axis=0)
    fall = _sc_fbuild(c1, c2, pe, sub, wpk)

    ss = sep_subgraphs.astype(_i32)
    srcs = jnp.concatenate([ss[:, 0], ss[:, 1]])
    dsts = jnp.concatenate([ss[:, 1], ss[:, 0]])
    degp = _sc_deg(dsts)
    aggall = _sc_agg(fall, srcs, dsts)

    q1, q2 = _tc_final(
        fall, aggall, degp, sums, sqs,
        gamma.reshape(1, F), beta.reshape(1, F),
        Wg1, bg1.reshape(1, F), Wg2, bg2.reshape(1, F),
        (V1a, V1ab.reshape(1, H), V1b, V1bb.reshape(1, H),
         V1c, V1cb.reshape(1, NCQ)),
        (V2a, V2ab.reshape(1, H), V2b, V2bb.reshape(1, H),
         V2c, V2cb.reshape(1, NCQ)))
    return q1, q2


# speedup vs baseline: 1.2893x; 1.2893x over previous
import functools

import jax
import jax.numpy as jnp
from jax import lax
from jax.experimental import pallas as pl
from jax.experimental.pallas import tpu as pltpu
from jax.experimental.pallas import tpu_sc as plsc

N = 10000
E = 320000
F = 128
NSUB = 4096
S = 16
M = NSUB * S
K2 = 2 * M
NCQ = 3
H = 1024
NCORE = 2
NSC = 16
NW = NCORE * NSC

_f32 = jnp.float32
_i32 = jnp.int32

_mesh = plsc.VectorSubcoreMesh(core_axis_name="c", subcore_axis_name="s")
_sc_params = pltpu.CompilerParams(needs_layout_passes=False)
_sc_params_nt = pltpu.CompilerParams(needs_layout_passes=False,
                                     use_tc_tiling_on_sc=False)


def _leaky(x):
    return jnp.maximum(x, x * 0.01)


def _z16f():
    return jnp.zeros((16,), _f32)


_EB1 = 2560


def _sc_scatter_body(xt, src_h, dst_h, a_out, xblk, ablk, srcb, dstb):
    c = lax.axis_index("c")
    s = lax.axis_index("s")
    w = c * NSC + s
    pltpu.sync_copy(xt.at[w], xblk)

    def zero(i, _):
        ablk[pl.ds(i * 16, 16)] = _z16f()
        return 0

    lax.fori_loop(0, (N * 4) // 16, zero, 0)

    def blk(b, _):
        pltpu.sync_copy(src_h.at[pl.ds(b * _EB1, _EB1)], srcb)
        pltpu.sync_copy(dst_h.at[pl.ds(b * _EB1, _EB1)], dstb)

        def inner(i, _):
            for u in range(2):
                s16 = srcb[pl.ds(i * 32 + u * 16, 16)] * 4
                d16 = dstb[pl.ds(i * 32 + u * 16, 16)] * 4
                for j in range(4):
                    v = plsc.load_gather(xblk, [s16 + j])
                    plsc.addupdate_scatter(ablk, [d16 + j], v)
            return 0

        lax.fori_loop(0, _EB1 // 32, inner, 0)
        return 0

    lax.fori_loop(0, E // _EB1, blk, 0)
    pltpu.sync_copy(ablk, a_out.at[w])


_sc_scatter = pl.kernel(
    _sc_scatter_body,
    out_type=jax.ShapeDtypeStruct((NW, N * 4), _f32),
    mesh=_mesh,
    compiler_params=_sc_params,
    scratch_types=[
        pltpu.VMEM((N * 4,), _f32),
        pltpu.VMEM((N * 4,), _f32),
        pltpu.VMEM((_EB1,), _i32),
        pltpu.VMEM((_EB1,), _i32),
    ],
)


def _tc_c_body(a_ref, x_ref, ws1, wd1, b1, ws2, wd2, b2, c1_ref, c2_ref):
    a = a_ref[...]
    x = x_ref[...]
    c1_ref[...] = (jnp.dot(a, ws1[...], preferred_element_type=_f32,
                           precision=lax.Precision.HIGHEST)
                   + jnp.dot(x, wd1[...], preferred_element_type=_f32,
                           precision=lax.Precision.HIGHEST)
                   + b1[...])
    c2_ref[...] = (jnp.dot(a, ws2[...], preferred_element_type=_f32,
                           precision=lax.Precision.HIGHEST)
                   + jnp.dot(x, wd2[...], preferred_element_type=_f32,
                           precision=lax.Precision.HIGHEST)
                   + b2[...])


def _tc_c(a, x, ws1, wd1, b1, ws2, wd2, b2):
    bn = 1000
    row = pl.BlockSpec((bn, F), lambda i: (i, 0))
    wsp = pl.BlockSpec((F, F), lambda i: (0, 0))
    bsp = pl.BlockSpec((1, F), lambda i: (0, 0))
    return pl.pallas_call(
        _tc_c_body,
        grid=(N // bn,),
        in_specs=[row, row, wsp, wsp, bsp, wsp, wsp, bsp],
        out_specs=[row, row],
        out_shape=[jax.ShapeDtypeStruct((N, F), _f32)] * 2,
    )(a, x, ws1, wd1, b1, ws2, wd2, b2)


_EB2 = 2560


def _sc_stats_body(ctall, wall, dst_h, ang_h, act_h, gte_h,
                   st_out, cblk, dstb, ab, qb, gb, wav, wqv, wgv, accv):
    c = lax.axis_index("c")
    s = lax.axis_index("s")
    pltpu.sync_copy(ctall.at[c * NSC + s], cblk)
    pltpu.sync_copy(wall.at[c * 3 + 0], wav)
    pltpu.sync_copy(wall.at[c * 3 + 1], wqv)
    pltpu.sync_copy(wall.at[c * 3 + 2], wgv)

    base = s * 8
    waf = [plsc.load_gather(wav, [jnp.full((16,), base + j, _i32)])
           for j in range(8)]
    wqf = [plsc.load_gather(wqv, [jnp.full((16,), base + j, _i32)])
           for j in range(8)]
    wgf = [plsc.load_gather(wgv, [jnp.full((16,), base + j, _i32)])
           for j in range(8)]

    def blk(b, carry):
        pltpu.sync_copy(dst_h.at[pl.ds(b * _EB2, _EB2)], dstb)
        pltpu.sync_copy(ang_h.at[pl.ds(b * _EB2, _EB2)], ab)
        pltpu.sync_copy(act_h.at[pl.ds(b * _EB2, _EB2)], qb)
        pltpu.sync_copy(gte_h.at[pl.ds(b * _EB2, _EB2)], gb)

        def inner(i, cr):
            sums = list(cr[:8])
            sqsl = list(cr[8:])
            for u in range(2):
                d16 = dstb[pl.ds(i * 32 + u * 16, 16)] * 8
                a16 = ab[pl.ds(i * 32 + u * 16, 16)]
                q16 = qb[pl.ds(i * 32 + u * 16, 16)]
                g16 = gb[pl.ds(i * 32 + u * 16, 16)]
                for j in range(8):
                    cj = plsc.load_gather(cblk, [d16 + j])
                    h = cj + a16 * waf[j] + q16 * wqf[j] + g16 * wgf[j]
                    v = _leaky(h)
                    sums[j] = sums[j] + v
                    sqsl[j] = sqsl[j] + v * v
            return tuple(sums) + tuple(sqsl)

        part = lax.fori_loop(0, _EB2 // 32, inner,
                             tuple(_z16f() for _ in range(16)))
        return tuple(c + p for c, p in zip(carry, part))

    init = tuple(_z16f() for _ in range(16))
    acc = lax.fori_loop(0, E // _EB2, blk, init)
    for j in range(16):
        sj = jnp.full((16,), jnp.sum(acc[j]))
        plsc.store_scatter(accv, [jnp.full((16,), j, _i32)], sj)
    pltpu.sync_copy(accv, st_out.at[c, pl.ds(s * 16, 16)])


_sc_stats = pl.kernel(
    _sc_stats_body,
    out_type=jax.ShapeDtypeStruct((2, 256), _f32),
    mesh=_mesh,
    compiler_params=_sc_params,
    scratch_types=[
        pltpu.VMEM((N * 8,), _f32),
        pltpu.VMEM((_EB2,), _i32),
        pltpu.VMEM((_EB2,), _f32),
        pltpu.VMEM((_EB2,), _f32),
        pltpu.VMEM((_EB2,), _f32),
        pltpu.VMEM((F,), _f32),
        pltpu.VMEM((F,), _f32),
        pltpu.VMEM((F,), _f32),
        pltpu.VMEM((16,), _f32),
    ],
)


_MT = M // NW
_GB = 128


def _sc_fbuild_body(c1_h, c2_h, pe_h, sub_h, wpk_h,
                    fall_out,
                    sgb, peb, dlist, c1b, c2b, ob1, ob2, wpk, sem):
    c = lax.axis_index("c")
    s = lax.axis_index("s")
    w = c * NSC + s
    base = w * _MT
    pltpu.sync_copy(sub_h.at[pl.ds(base, _MT)], sgb)
    pltpu.sync_copy(wpk_h, wpk)
    descs = [
        pltpu.async_copy(pe_h.at[sgb.at[pl.ds(ch * _GB, _GB)]],
                         peb.at[pl.ds(ch * _GB, _GB)], sem)
        for ch in range(_MT // _GB)
    ]
    for d in descs:
        d.wait()

    lane = jax.lax.iota(_i32, 16)

    def blk(b, _):
        for t in range(_GB // 16):
            r16 = b * _GB + t * 16 + lane
            dstf = plsc.load_gather(peb, [r16, jnp.zeros((16,), _i32)])
            dlist[pl.ds(t * 16, 16)] = plsc.bitcast(dstf, _i32)
        d1 = pltpu.async_copy(c1_h.at[dlist], c1b, sem)
        d2 = pltpu.async_copy(c2_h.at[dlist], c2b, sem)
        d1.wait()
        d2.wait()

        for lp in range(2):
            cb = (c1b, c2b)[lp]
            ob = (ob1, ob2)[lp]
            wv = [[wpk[3 * lp + r, pl.ds(j * 16, 16)] for j in range(8)]
                  for r in range(3)]

            def edge(i, _):
                row = b * _GB + i
                a16 = plsc.load_gather(
                    peb, [jnp.full((16,), row, _i32), jnp.full((16,), 1, _i32)])
                q16 = plsc.load_gather(
                    peb, [jnp.full((16,), row, _i32), jnp.full((16,), 2, _i32)])
                g16 = plsc.load_gather(
                    peb, [jnp.full((16,), row, _i32), jnp.full((16,), 3, _i32)])
                for j in range(8):
                    cj = cb[i, pl.ds(j * 16, 16)]
                    h = cj + a16 * wv[0][j] + q16 * wv[1][j] + g16 * wv[2][j]
                    ob[i, pl.ds(j * 16, 16)] = _leaky(h)
                return 0

            lax.fori_loop(0, _GB, edge, 0)

        rb = base + b * _GB
        pltpu.sync_copy(ob1, fall_out.at[pl.ds(rb, _GB)])
        pltpu.sync_copy(ob2, fall_out.at[pl.ds(M + rb, _GB)])
        return 0

    lax.fori_loop(0, _MT // _GB, blk, 0)


def _edge_rows_spec(shape):
    return pltpu.VMEM(shape, _f32)


_sc_fbuild = pl.kernel(
    _sc_fbuild_body,
    out_type=jax.ShapeDtypeStruct((2 * M, F), _f32),
    mesh=_mesh,
    compiler_params=_sc_params_nt,
    scratch_types=[
        pltpu.VMEM((_MT,), _i32),
        pltpu.VMEM((_MT, 16), _f32),
        pltpu.VMEM((_GB,), _i32),
        pltpu.VMEM((_GB, F), _f32),
        pltpu.VMEM((_GB, F), _f32),
        pltpu.VMEM((_GB, F), _f32),
        pltpu.VMEM((_GB, F), _f32),
        pltpu.VMEM((8, F), _f32),
        pltpu.SemaphoreType.DMA,
    ],
)


_DT = K2 // NW


def _sc_deg_body(dst_h, deg_out, degb, sepb):
    c = lax.axis_index("c")
    s = lax.axis_index("s")
    w = c * NSC + s

    def zero(i, _):
        degb[pl.ds(i * 16, 16)] = _z16f()
        return 0

    lax.fori_loop(0, M // 16, zero, 0)
    pltpu.sync_copy(dst_h.at[pl.ds(w * _DT, _DT)], sepb)
    ones = jnp.ones((16,), _f32)

    def hist(i, _):
        d16 = sepb[pl.ds(i * 16, 16)]
        plsc.addupdate_scatter(degb, [d16], ones)
        return 0

    lax.fori_loop(0, _DT // 16, hist, 0)
    pltpu.sync_copy(degb, deg_out.at[w])


_sc_deg = pl.kernel(
    _sc_deg_body,
    out_type=jax.ShapeDtypeStruct((NW, M), _f32),
    mesh=_mesh,
    compiler_params=_sc_params,
    scratch_types=[
        pltpu.VMEM((M,), _f32),
        pltpu.VMEM((_DT,), _i32),
    ],
)


_AT = K2 // NSC
_CH = 8192
_CHP = _CH + 256
_NP = M // _CH


def _sc_agg_body(fall_h, src_h, dst_h, aggall_out,
                 srcb, dstb, selS, selD, gsrc, gdst, rowbuf, zb, sem, acc):
    c = lax.axis_index("c")
    s = lax.axis_index("s")
    ebase = s * _AT
    pltpu.sync_copy(src_h.at[pl.ds(ebase, _AT)], srcb)
    pltpu.sync_copy(dst_h.at[pl.ds(ebase, _AT)], dstb)

    def zzb(k, _):
        for t in range(8):
            zb[k, pl.ds(t * 16, 16)] = _z16f()
        return 0

    lax.fori_loop(0, 16, zzb, 0)

    rows_per_tile = _CHP // NSC
    coff = c * M

    for p in range(_NP):
        def zr(k, _):
            pltpu.sync_copy(zb, acc.at[pl.ds(s * rows_per_tile + k * 16, 16)])
            return 0

        lax.fori_loop(0, rows_per_tile // 16, zr, 0)
        plsc.subcore_barrier()

        lo = p * _CH

        def scan(i, off):
            d16 = dstb[pl.ds(i * 16, 16)] - lo
            s16 = srcb[pl.ds(i * 16, 16)]
            m = (d16 >= 0) & (d16 < _CH)
            plsc.store_compressed(selS.at[pl.ds(off, 16)], s16, mask=m)
            plsc.store_compressed(selD.at[pl.ds(off, 16)], d16, mask=m)
            return off + jnp.sum(jnp.where(m, 1, 0))

        off = lax.fori_loop(0, _AT // 16, scan, 0)

        def pad(k, _):
            selD[pl.ds(off + k * 16, 16)] = jnp.full((16,), _CH, _i32)
            selS[pl.ds(off + k * 16, 16)] = jnp.zeros((16,), _i32)
            return 0

        lax.fori_loop(0, _GB // 16, pad, 0)
        nblk = (off + _GB - 1) // _GB

        def gs(b, _):
            for t in range(_GB // 16):
                sl = pl.ds(t * 16, 16)
                gsrc[sl] = selS[pl.ds(b * _GB + t * 16, 16)] + coff
                gdst[sl] = selD[pl.ds(b * _GB + t * 16, 16)]

            pltpu.async_copy(fall_h.at[gsrc], rowbuf, sem).wait()
            pltpu.sync_copy(rowbuf, acc.at[gdst], add=True)
            return 0

        lax.fori_loop(0, nblk, gs, 0)
        plsc.subcore_barrier()

        wrow = s * (_CH // NSC)

        def wb(k, _):
            rsl = pl.ds(wrow + k * _GB, _GB)
            pltpu.sync_copy(acc.at[rsl], rowbuf)
            pltpu.sync_copy(
                rowbuf,
                aggall_out.at[pl.ds(coff + lo + wrow + k * _GB, _GB)])
            return 0

        lax.fori_loop(0, (_CH // NSC) // _GB, wb, 0)
        plsc.subcore_barrier()


_sc_agg = pl.kernel(
    _sc_agg_body,
    out_type=jax.ShapeDtypeStruct((2 * M, F), _f32),
    mesh=_mesh,
    compiler_params=_sc_params,
    scratch_types=[
        pltpu.VMEM((_AT,), _i32),
        pltpu.VMEM((_AT,), _i32),
        pltpu.VMEM((_AT + _GB,), _i32),
        pltpu.VMEM((_AT + _GB,), _i32),
        pltpu.VMEM((_GB,), _i32),
        pltpu.VMEM((_GB,), _i32),
        pltpu.VMEM((_GB, F), _f32),
        pltpu.VMEM((16, F), _f32),
        pltpu.SemaphoreType.DMA,
        pltpu.VMEM_SHARED((_CHP, F), _f32),
    ],
)


def _tc_fin_body(f1, f2, a1, a2, dgp, sums, sqs, gma, bta,
                 wg1, bg1, wg2, bg2,
                 v1a, v1ab, v1b, v1bb, v1c, v1cb,
                 v2a, v2ab, v2b, v2bb, v2c, v2cb,
                 q1_ref, q2_ref):
    mu = sums[...] / E
    var = sqs[...] / E - mu * mu
    r = gma[...] / jnp.sqrt(var + 1e-5)
    sh = bta[...] - mu * r
    deg = jnp.sum(dgp[...], axis=0)
    rec = 1.0 / (deg + 1.0)
    s2c = (1.0 + deg * rec)[:, None]

    def head(fv, av, ri, shi, wg, bg, va, vab, vb, vbb, vc, vcb, out_ref):
        t = ri * (fv[...] + av[...] * rec[:, None]) + shi * s2c
        g = _leaky(jnp.dot(t, wg[...], preferred_element_type=_f32,
                           precision=lax.Precision.HIGHEST) + bg[...])
        mrows = g.shape[0] // S
        gm = jnp.mean(g.reshape(mrows, S, F), axis=1)
        z = _leaky(jnp.dot(gm, va[...], preferred_element_type=_f32,
                           precision=lax.Precision.HIGHEST) + vab[...])
        z = _leaky(jnp.dot(z, vb[...], preferred_element_type=_f32,
                           precision=lax.Precision.HIGHEST) + vbb[...])
        out_ref[...] = jnp.dot(z, vc[...], preferred_element_type=_f32,
                           precision=lax.Precision.HIGHEST) + vcb[...]

    head(f1, a1, r[0:1, :], sh[0:1, :], wg1, bg1,
         v1a, v1ab, v1b, v1bb, v1c, v1cb, q1_ref)
    head(f2, a2, r[1:2, :], sh[1:2, :], wg2, bg2,
         v2a, v2ab, v2b, v2bb, v2c, v2cb, q2_ref)


def _tc_final(fall, aggall, degp, sums, sqs, gma, bta,
              wg1, bg1, wg2, bg2, v1, v2):
    R = 2048
    grid = (M // R,)
    row = pl.BlockSpec((R, F), lambda i: (i, 0))
    row2 = pl.BlockSpec((R, F), lambda i: (i + M // R, 0))
    dsp = pl.BlockSpec((NW, R), lambda i: (0, i))
    c2f = pl.BlockSpec((2, F), lambda i: (0, 0))
    c1f = pl.BlockSpec((1, F), lambda i: (0, 0))
    wsp = pl.BlockSpec((F, F), lambda i: (0, 0))
    vaspec = pl.BlockSpec((F, H), lambda i: (0, 0))
    vbspec = pl.BlockSpec((H, H), lambda i: (0, 0))
    vcspec = pl.BlockSpec((H, NCQ), lambda i: (0, 0))
    h1 = pl.BlockSpec((1, H), lambda i: (0, 0))
    c1n = pl.BlockSpec((1, NCQ), lambda i: (0, 0))
    osp = pl.BlockSpec((R // S, NCQ), lambda i: (i, 0))
    v1a, v1ab, v1b, v1bb, v1c, v1cb = v1
    v2a, v2ab, v2b, v2bb, v2c, v2cb = v2
    return pl.pallas_call(
        _tc_fin_body,
        grid=grid,
        in_specs=[row, row2, row, row2, dsp, c2f, c2f, c1f, c1f,
                  wsp, c1f, wsp, c1f,
                  vaspec, h1, vbspec, h1, vcspec, c1n,
                  vaspec, h1, vbspec, h1, vcspec, c1n],
        out_specs=[osp, osp],
        out_shape=[jax.ShapeDtypeStruct((NSUB, NCQ), _f32)] * 2,
    )(fall, fall, aggall, aggall, degp, sums, sqs, gma, bta,
      wg1, bg1, wg2, bg2,
      v1a, v1ab, v1b, v1bb, v1c, v1cb,
      v2a, v2ab, v2b, v2bb, v2c, v2cb)


def kernel(node_features, actions, edge_index, angles, sub_graphs,
           sep_subgraphs, gt_edges, post_input,
           Wsrc1, Wdst1, wa1, wq1, wg1, b1,
           Wsrc2, Wdst2, wa2, wq2, wg2, b2,
           gamma, beta, Wg1, bg1, Wg2, bg2,
           V1a, V1ab, V1b, V1bb, V1c, V1cb,
           V2a, V2ab, V2b, V2bb, V2c, V2cb):
    x = node_features.astype(_f32)
    src = edge_index[0].astype(_i32)
    dst = edge_index[1].astype(_i32)
    sub = sub_graphs.astype(_i32)

    xt = x.reshape(N, NW, 4).transpose(1, 0, 2).reshape(NW, N * 4)
    a_out = _sc_scatter(xt, src, dst)
    A = a_out.reshape(NW, N, 4).transpose(1, 0, 2).reshape(N, F)

    c1, c2 = _tc_c(A, x, Wsrc1, Wdst1, b1.reshape(1, F),
                   Wsrc2, Wdst2, b2.reshape(1, F))

    ct1 = c1.reshape(N, NSC, 8).transpose(1, 0, 2).reshape(NSC, N * 8)
    ct2 = c2.reshape(N, NSC, 8).transpose(1, 0, 2).reshape(NSC, N * 8)
    ctall = jnp.concatenate([ct1, ct2], axis=0)
    wall = jnp.stack([wa1, wq1, wg1, wa2, wq2, wg2])
    st = _sc_stats(ctall, wall, dst, angles, actions, gt_edges)
    stv = st.reshape(2, NSC, 2, 8)
    sums = stv[:, :, 0, :].reshape(2, F)
    sqs = stv[:, :, 1, :].reshape(2, F)

    dstf = lax.bitcast_convert_type(dst, _f32)
    pe = jnp.concatenate(
        [dstf[:, None], angles[:, None], actions[:, None], gt_edges[:, None],
         jnp.zeros((E, 12), _f32)], axis=1)
    wpk = jnp.concatenate(
        [wa1[None], wq1[None], wg1[None], wa2[None], wq2[None], wg2[None],
         jnp.zeros((2, F), _f32)], axis=0)
    fall = _sc_fbuild(c1, c2, pe, sub, wpk)

    ss = sep_subgraphs.astype(_i32)
    srcs = jnp.concatenate([ss[:, 0], ss[:, 1]])
    dsts = jnp.concatenate([ss[:, 1], ss[:, 0]])
    degp = _sc_deg(dsts)
    aggall = _sc_agg(fall, srcs, dsts)

    q1, q2 = _tc_final(
        fall, aggall, degp, sums, sqs,
        gamma.reshape(1, F), beta.reshape(1, F),
        Wg1, bg1.reshape(1, F), Wg2, bg2.reshape(1, F),
        (V1a, V1ab.reshape(1, H), V1b, V1bb.reshape(1, H),
         V1c, V1cb.reshape(1, NCQ)),
        (V2a, V2ab.reshape(1, H), V2b, V2bb.reshape(1, H),
         V2c, V2cb.reshape(1, NCQ)))
    return q1, q2
